# Initial kernel scaffold; baseline (speedup 1.0000x reference)
import jax
import jax.numpy as jnp
from jax.experimental import pallas as pl


def kernel(x, edge_index, batch, W1, b1, W2, b2, Wfc1, bfc1, Wfc2, bfc2):
    raise NotImplementedError("write your pallas kernel here")


# speedup vs baseline: 15.2863x; 15.2863x over previous
import functools

import jax
import jax.numpy as jnp
from jax import lax
from jax.experimental import pallas as pl
from jax.experimental.pallas import tpu as pltpu
from jax.experimental.pallas import tpu_sc as plsc

N0 = 10000
E0 = 320000
D = 128
H = 128
O_ = 10
G = 64

NC, NS = 2, 16
NW = NC * NS
BN = 1024
NP = 10240
ROWS_PT = NP // NS
C = 128
CPW = 79
EP = NW * CPW * C
NGRID = NP // BN

_f32 = jnp.float32
_MESH = plsc.VectorSubcoreMesh(core_axis_name="c", subcore_axis_name="s")


def _fill(buf, val):
    def row(r, carry):
        for k in range(H // 16):
            buf[r, pl.ds(k * 16, 16)] = jnp.full((16,), val, _f32)
        return carry
    lax.fori_loop(0, C, row, 0)


def _zero_acc(acc, zbuf, t0):
    def cp(j, carry):
        pltpu.sync_copy(zbuf, acc.at[pl.ds(t0 + j * C, C)])
        return carry
    lax.fori_loop(0, ROWS_PT // C, cp, 0)


def _deg_body(dst_hbm, out_hbm, acc, buf, didx):
    c = lax.axis_index("c")
    s = lax.axis_index("s")
    wid = s * NC + c
    t0 = s * ROWS_PT
    _fill(buf, 0.0)
    _zero_acc(acc, buf, t0)
    _fill(buf, 1.0)
    plsc.subcore_barrier()
    base = wid * (CPW * C)
    def step(j, carry):
        off = base + j * C
        pltpu.sync_copy(dst_hbm.at[pl.ds(off, C)], didx)
        pltpu.sync_copy(buf, acc.at[didx], add=True)
        return carry
    lax.fori_loop(0, CPW, step, 0)
    plsc.subcore_barrier()
    pltpu.sync_copy(acc.at[pl.ds(t0, ROWS_PT)],
                    out_hbm.at[c, pl.ds(t0, ROWS_PT)])


_deg_call = pl.kernel(
    _deg_body,
    out_type=jax.ShapeDtypeStruct((NC, NP, H), _f32),
    mesh=_MESH,
    scratch_types=[
        pltpu.VMEM_SHARED((NP, H), _f32),
        pltpu.VMEM((C, H), _f32),
        pltpu.VMEM((C,), jnp.int32),
    ],
)


def _agg_body(g_hbm, src_hbm, dst_hbm, out_hbm, acc, zbuf, sidx, didx, rows,
              gsem):
    c = lax.axis_index("c")
    s = lax.axis_index("s")
    wid = s * NC + c
    t0 = s * ROWS_PT
    _fill(zbuf, 0.0)
    _zero_acc(acc, zbuf, t0)
    plsc.subcore_barrier()
    base = wid * (CPW * C)
    def step(j, carry):
        off = base + j * C
        pltpu.sync_copy(src_hbm.at[pl.ds(off, C)], sidx)
        pltpu.sync_copy(dst_hbm.at[pl.ds(off, C)], didx)
        pltpu.async_copy(g_hbm.at[sidx], rows, gsem).wait()
        pltpu.sync_copy(rows, acc.at[didx], add=True)
        return carry
    lax.fori_loop(0, CPW, step, 0)
    plsc.subcore_barrier()
    pltpu.sync_copy(acc.at[pl.ds(t0, ROWS_PT)],
                    out_hbm.at[c, pl.ds(t0, ROWS_PT)])


_agg_call = pl.kernel(
    _agg_body,
    out_type=jax.ShapeDtypeStruct((NC, NP, H), _f32),
    mesh=_MESH,
    scratch_types=[
        pltpu.VMEM_SHARED((NP, H), _f32),
        pltpu.VMEM((C, H), _f32),
        pltpu.VMEM((C,), jnp.int32),
        pltpu.VMEM((C,), jnp.int32),
        pltpu.VMEM((C, H), _f32),
        pltpu.SemaphoreType.DMA,
    ],
)


def _k2_body(x_ref, w_ref, dp_ref, g_ref, dis_ref):
    deg = dp_ref[0, :, 0:1] + dp_ref[1, :, 0:1] + 1.0
    dis = lax.rsqrt(deg)
    m = jnp.dot(x_ref[...], w_ref[...], preferred_element_type=_f32)
    g_ref[...] = dis * m
    dis_ref[...] = jnp.broadcast_to(dis, (BN, 16))


_k2_call = pl.pallas_call(
    _k2_body,
    grid=(NGRID,),
    in_specs=[
        pl.BlockSpec((BN, D), lambda i: (i, 0)),
        pl.BlockSpec((D, H), lambda i: (0, 0)),
        pl.BlockSpec((NC, BN, H), lambda i: (0, i, 0)),
    ],
    out_specs=[
        pl.BlockSpec((BN, H), lambda i: (i, 0)),
        pl.BlockSpec((BN, 16), lambda i: (i, 0)),
    ],
    out_shape=[
        jax.ShapeDtypeStruct((NP, H), _f32),
        jax.ShapeDtypeStruct((NP, 16), _f32),
    ],
)


def _k4_body(a_ref, g1_ref, dis_ref, b1_ref, w2_ref, g2_ref):
    dis = dis_ref[:, 0:1]
    t = dis * (a_ref[0] + a_ref[1] + g1_ref[...]) + b1_ref[...]
    h1 = jnp.maximum(t, 0.0)
    g2_ref[...] = dis * jnp.dot(h1, w2_ref[...], preferred_element_type=_f32)


_k4_call = pl.pallas_call(
    _k4_body,
    grid=(NGRID,),
    in_specs=[
        pl.BlockSpec((NC, BN, H), lambda i: (0, i, 0)),
        pl.BlockSpec((BN, H), lambda i: (i, 0)),
        pl.BlockSpec((BN, 16), lambda i: (i, 0)),
        pl.BlockSpec((1, H), lambda i: (0, 0)),
        pl.BlockSpec((H, H), lambda i: (0, 0)),
    ],
    out_specs=pl.BlockSpec((BN, H), lambda i: (i, 0)),
    out_shape=jax.ShapeDtypeStruct((NP, H), _f32),
)


def _k6_body(a_ref, g2_ref, dis_ref, b2_ref, batch_ref, wf1_ref, bf1_ref,
             wf2_ref, bf2_ref, out_ref, sums, cnts):
    i = pl.program_id(0)

    @pl.when(i == 0)
    def _init():
        sums[...] = jnp.zeros((G, H), _f32)
        cnts[...] = jnp.zeros((G, H), _f32)

    dis = dis_ref[:, 0:1]
    t = dis * (a_ref[0] + a_ref[1] + g2_ref[...]) + b2_ref[...]
    h2 = jnp.maximum(t, 0.0)
    gid = lax.broadcasted_iota(jnp.int32, (G, BN), 0)
    mask = (gid == batch_ref[...]).astype(_f32)
    sums[...] += jnp.dot(mask, h2, preferred_element_type=_f32)
    cnts[...] += jnp.broadcast_to(
        jnp.sum(mask, axis=1, keepdims=True), (G, H))

    @pl.when(i == NGRID - 1)
    def _fin():
        pooled = sums[...] / jnp.maximum(cnts[...], 1.0)
        fc1 = jnp.maximum(
            jnp.dot(pooled, wf1_ref[...], preferred_element_type=_f32)
            + bf1_ref[...], 0.0)
        logits = (jnp.dot(fc1, wf2_ref[...], preferred_element_type=_f32)
                  + bf2_ref[...])
        m = jnp.max(logits, axis=1, keepdims=True)
        e = jnp.exp(logits - m)
        out_ref[...] = e / jnp.sum(e, axis=1, keepdims=True)


_k6_call = pl.pallas_call(
    _k6_body,
    grid=(NGRID,),
    in_specs=[
        pl.BlockSpec((NC, BN, H), lambda i: (0, i, 0)),
        pl.BlockSpec((BN, H), lambda i: (i, 0)),
        pl.BlockSpec((BN, 16), lambda i: (i, 0)),
        pl.BlockSpec((1, H), lambda i: (0, 0)),
        pl.BlockSpec((1, BN), lambda i: (0, i)),
        pl.BlockSpec((H, H), lambda i: (0, 0)),
        pl.BlockSpec((1, H), lambda i: (0, 0)),
        pl.BlockSpec((H, O_), lambda i: (0, 0)),
        pl.BlockSpec((1, O_), lambda i: (0, 0)),
    ],
    out_specs=pl.BlockSpec((G, O_), lambda i: (0, 0)),
    out_shape=jax.ShapeDtypeStruct((G, O_), _f32),
    scratch_shapes=[
        pltpu.VMEM((G, H), _f32),
        pltpu.VMEM((G, H), _f32),
    ],
)


def kernel(x, edge_index, batch, W1, b1, W2, b2, Wfc1, bfc1, Wfc2, bfc2):
    src = edge_index[0]
    dst = edge_index[1]
    pad = EP - E0
    pidx = jnp.arange(pad, dtype=jnp.int32)
    srcp = jnp.concatenate([src, (pidx * 97) % N0])
    dstp = jnp.concatenate([dst, N0 + pidx % (NP - N0)])
    xp = jnp.zeros((NP, D), _f32).at[:N0].set(x)
    batchp = jnp.full((1, NP), G, jnp.int32).at[0, :N0].set(batch)

    degp = _deg_call(dstp)
    g1, dis16 = _k2_call(xp, W1, degp)
    agg1 = _agg_call(g1, srcp, dstp)
    g2 = _k4_call(agg1, g1, dis16, b1.reshape(1, H), W2)
    agg2 = _agg_call(g2, srcp, dstp)
    return _k6_call(agg2, g2, dis16, b2.reshape(1, H), batchp,
                    Wfc1, bfc1.reshape(1, H), Wfc2, bfc2.reshape(1, O_))


# speedup vs baseline: 28.9441x; 1.8935x over previous
import functools

import jax
import jax.numpy as jnp
from jax import lax
from jax.experimental import pallas as pl
from jax.experimental.pallas import tpu as pltpu
from jax.experimental.pallas import tpu_sc as plsc

N0 = 10000
E0 = 320000
D = 128
H = 128
O_ = 10
G = 64

NC, NS = 2, 16
NW = NC * NS
BN = 1024
NP = 10240
ROWS_PT = NP // NS
C = 128
CPW = 80
NB = 4
EP = NW * CPW * C
NCH = EP // C
NGRID = NP // BN

_f32 = jnp.float32
_MESH = plsc.VectorSubcoreMesh(core_axis_name="c", subcore_axis_name="s")


def _fill(buf, val):
    def row(r, carry):
        for k in range(H // 16):
            buf[r, pl.ds(k * 16, 16)] = jnp.full((16,), val, _f32)
        return carry
    lax.fori_loop(0, C, row, 0)


def _zero_acc(acc, zbuf, t0):
    def cp(j, carry):
        pltpu.sync_copy(zbuf, acc.at[pl.ds(t0 + j * C, C)])
        return carry
    lax.fori_loop(0, ROWS_PT // C, cp, 0)


def _deg_body(sd_hbm, out_hbm, acc, buf, islab, isem):
    c = lax.axis_index("c")
    s = lax.axis_index("s")
    wid = s * NC + c
    t0 = s * ROWS_PT
    idesc = pltpu.async_copy(sd_hbm.at[pl.ds(wid * CPW, CPW)], islab, isem)
    _fill(buf, 0.0)
    _zero_acc(acc, buf, t0)
    _fill(buf, 1.0)
    plsc.subcore_barrier()
    idesc.wait()
    def step(j, carry):
        pltpu.sync_copy(buf, acc.at[islab.at[j, 1]], add=True)
        return carry
    lax.fori_loop(0, CPW, step, 0)
    plsc.subcore_barrier()
    pltpu.sync_copy(acc.at[pl.ds(t0, ROWS_PT)],
                    out_hbm.at[c, pl.ds(t0, ROWS_PT)])


_deg_call = pl.kernel(
    _deg_body,
    out_type=jax.ShapeDtypeStruct((NC, NP, H), _f32),
    mesh=_MESH,
    scratch_types=[
        pltpu.VMEM_SHARED((NP, H), _f32),
        pltpu.VMEM((C, H), _f32),
        pltpu.VMEM((CPW, 2, C), jnp.int32),
        pltpu.SemaphoreType.DMA,
    ],
)


def _agg_body(g_hbm, sd_hbm, out_hbm, acc, r0, r1, i0, i1, i2, i3,
              gs0, gs1, ss0, ss1, is0, is1, is2, is3):
    c = lax.axis_index("c")
    s = lax.axis_index("s")
    wid = s * NC + c
    t0 = s * ROWS_PT
    rowss = [r0, r1]
    idxs = [i0, i1, i2, i3]
    gsems = [gs0, gs1]
    ssems = [ss0, ss1]
    isems = [is0, is1, is2, is3]
    _fill(r0, 0.0)
    _zero_acc(acc, r0, t0)
    plsc.subcore_barrier()
    cb = wid * CPW
    for b in range(2):
        pltpu.sync_copy(sd_hbm.at[cb + b], idxs[b])
        pltpu.async_copy(g_hbm.at[idxs[b].at[0]], rowss[b], gsems[b])
    for b in range(2, 4):
        pltpu.async_copy(sd_hbm.at[cb + b], idxs[b], isems[b])

    def outer(i, carry):
        for b in range(4):
            j = i * 4 + b
            rb = b % 2
            pltpu.make_async_copy(
                g_hbm.at[idxs[b].at[0]], rowss[rb], gsems[rb]).wait()
            pltpu.async_copy(
                rowss[rb], acc.at[idxs[b].at[1]], ssems[rb], add=True)
            jn = j + 2

            @pl.when(jn < CPW)
            def _prefetch():
                pltpu.make_async_copy(
                    rowss[rb], acc.at[idxs[b].at[1]], ssems[rb]).wait()
                bn = (b + 2) % 4
                pltpu.make_async_copy(
                    sd_hbm.at[cb + jn], idxs[bn], isems[bn]).wait()
                pltpu.async_copy(g_hbm.at[idxs[bn].at[0]], rowss[rb],
                                 gsems[rb])
            jf = j + 4

            @pl.when(jf < CPW)
            def _ipf():
                pltpu.async_copy(sd_hbm.at[cb + jf], idxs[b], isems[b])
        return carry
    lax.fori_loop(0, CPW // 4, outer, 0)
    pltpu.make_async_copy(rowss[0], acc.at[idxs[2].at[1]], ssems[0]).wait()
    pltpu.make_async_copy(rowss[1], acc.at[idxs[3].at[1]], ssems[1]).wait()
    plsc.subcore_barrier()
    pltpu.sync_copy(acc.at[pl.ds(t0, ROWS_PT)],
                    out_hbm.at[c, pl.ds(t0, ROWS_PT)])


_agg_call = pl.kernel(
    _agg_body,
    out_type=jax.ShapeDtypeStruct((NC, NP, H), _f32),
    mesh=_MESH,
    scratch_types=[
        pltpu.VMEM_SHARED((NP, H), _f32),
        pltpu.VMEM((C, H), _f32),
        pltpu.VMEM((C, H), _f32),
        pltpu.VMEM((2, C), jnp.int32),
        pltpu.VMEM((2, C), jnp.int32),
        pltpu.VMEM((2, C), jnp.int32),
        pltpu.VMEM((2, C), jnp.int32),
        pltpu.SemaphoreType.DMA,
        pltpu.SemaphoreType.DMA,
        pltpu.SemaphoreType.DMA,
        pltpu.SemaphoreType.DMA,
        pltpu.SemaphoreType.DMA,
        pltpu.SemaphoreType.DMA,
        pltpu.SemaphoreType.DMA,
        pltpu.SemaphoreType.DMA,
    ],
)


def _k2_body(x_ref, w_ref, dp_ref, g_ref, dis_ref):
    deg = dp_ref[0, :, 0:1] + dp_ref[1, :, 0:1] + 1.0
    dis = lax.rsqrt(deg)
    m = jnp.dot(x_ref[...], w_ref[...], preferred_element_type=_f32)
    g_ref[...] = dis * m
    dis_ref[...] = jnp.broadcast_to(dis, (BN, 16))


_k2_call = pl.pallas_call(
    _k2_body,
    grid=(NGRID,),
    in_specs=[
        pl.BlockSpec((BN, D), lambda i: (i, 0)),
        pl.BlockSpec((D, H), lambda i: (0, 0)),
        pl.BlockSpec((NC, BN, H), lambda i: (0, i, 0)),
    ],
    out_specs=[
        pl.BlockSpec((BN, H), lambda i: (i, 0)),
        pl.BlockSpec((BN, 16), lambda i: (i, 0)),
    ],
    out_shape=[
        jax.ShapeDtypeStruct((NP, H), _f32),
        jax.ShapeDtypeStruct((NP, 16), _f32),
    ],
)


def _k4_body(a_ref, g1_ref, dis_ref, b1_ref, w2_ref, g2_ref):
    dis = dis_ref[:, 0:1]
    t = dis * (a_ref[0] + a_ref[1] + g1_ref[...]) + b1_ref[...]
    h1 = jnp.maximum(t, 0.0)
    g2_ref[...] = dis * jnp.dot(h1, w2_ref[...], preferred_element_type=_f32)


_k4_call = pl.pallas_call(
    _k4_body,
    grid=(NGRID,),
    in_specs=[
        pl.BlockSpec((NC, BN, H), lambda i: (0, i, 0)),
        pl.BlockSpec((BN, H), lambda i: (i, 0)),
        pl.BlockSpec((BN, 16), lambda i: (i, 0)),
        pl.BlockSpec((1, H), lambda i: (0, 0)),
        pl.BlockSpec((H, H), lambda i: (0, 0)),
    ],
    out_specs=pl.BlockSpec((BN, H), lambda i: (i, 0)),
    out_shape=jax.ShapeDtypeStruct((NP, H), _f32),
)


def _k6_body(a_ref, g2_ref, dis_ref, b2_ref, batch_ref, wf1_ref, bf1_ref,
             wf2_ref, bf2_ref, out_ref, sums, cnts):
    i = pl.program_id(0)

    @pl.when(i == 0)
    def _init():
        sums[...] = jnp.zeros((G, H), _f32)
        cnts[...] = jnp.zeros((G, H), _f32)

    dis = dis_ref[:, 0:1]
    t = dis * (a_ref[0] + a_ref[1] + g2_ref[...]) + b2_ref[...]
    h2 = jnp.maximum(t, 0.0)
    gid = lax.broadcasted_iota(jnp.int32, (G, BN), 0)
    mask = (gid == batch_ref[...]).astype(_f32)
    sums[...] += jnp.dot(mask, h2, preferred_element_type=_f32)
    cnts[...] += jnp.broadcast_to(
        jnp.sum(mask, axis=1, keepdims=True), (G, H))

    @pl.when(i == NGRID - 1)
    def _fin():
        pooled = sums[...] / jnp.maximum(cnts[...], 1.0)
        fc1 = jnp.maximum(
            jnp.dot(pooled, wf1_ref[...], preferred_element_type=_f32)
            + bf1_ref[...], 0.0)
        logits = (jnp.dot(fc1, wf2_ref[...], preferred_element_type=_f32)
                  + bf2_ref[...])
        m = jnp.max(logits, axis=1, keepdims=True)
        e = jnp.exp(logits - m)
        out_ref[...] = e / jnp.sum(e, axis=1, keepdims=True)


_k6_call = pl.pallas_call(
    _k6_body,
    grid=(NGRID,),
    in_specs=[
        pl.BlockSpec((NC, BN, H), lambda i: (0, i, 0)),
        pl.BlockSpec((BN, H), lambda i: (i, 0)),
        pl.BlockSpec((BN, 16), lambda i: (i, 0)),
        pl.BlockSpec((1, H), lambda i: (0, 0)),
        pl.BlockSpec((1, BN), lambda i: (0, i)),
        pl.BlockSpec((H, H), lambda i: (0, 0)),
        pl.BlockSpec((1, H), lambda i: (0, 0)),
        pl.BlockSpec((H, O_), lambda i: (0, 0)),
        pl.BlockSpec((1, O_), lambda i: (0, 0)),
    ],
    out_specs=pl.BlockSpec((G, O_), lambda i: (0, 0)),
    out_shape=jax.ShapeDtypeStruct((G, O_), _f32),
    scratch_shapes=[
        pltpu.VMEM((G, H), _f32),
        pltpu.VMEM((G, H), _f32),
    ],
)


def kernel(x, edge_index, batch, W1, b1, W2, b2, Wfc1, bfc1, Wfc2, bfc2):
    src = edge_index[0]
    dst = edge_index[1]
    pad = EP - E0
    pidx = jnp.arange(pad, dtype=jnp.int32)
    srcp = jnp.concatenate([src, (pidx * 97) % N0])
    dstp = jnp.concatenate([dst, N0 + pidx % (NP - N0)])
    sdp = jnp.stack([srcp.reshape(NCH, C), dstp.reshape(NCH, C)], axis=1)
    xp = jnp.zeros((NP, D), _f32).at[:N0].set(x)
    batchp = jnp.full((1, NP), G, jnp.int32).at[0, :N0].set(batch)

    degp = _deg_call(sdp)
    g1, dis16 = _k2_call(xp, W1, degp)
    agg1 = _agg_call(g1, sdp)
    g2 = _k4_call(agg1, g1, dis16, b1.reshape(1, H), W2)
    agg2 = _agg_call(g2, sdp)
    return _k6_call(agg2, g2, dis16, b2.reshape(1, H), batchp,
                    Wfc1, bfc1.reshape(1, H), Wfc2, bfc2.reshape(1, O_))


# speedup vs baseline: 35.4015x; 1.2231x over previous
import functools

import jax
import jax.numpy as jnp
from jax import lax
from jax.experimental import pallas as pl
from jax.experimental.pallas import tpu as pltpu
from jax.experimental.pallas import tpu_sc as plsc

N0 = 10000
E0 = 320000
D = 128
H = 128
O_ = 10
G = 64

NC, NS = 2, 16
NW = NC * NS
BN = 1024
NP = 10240
ROWS_PT = NP // NS
C = 128
CPW = 80
EP = NW * CPW * C
NCH = EP // C
NGRID = NP // BN

_f32 = jnp.float32
_MESH = plsc.VectorSubcoreMesh(core_axis_name="c", subcore_axis_name="s")


def _fill(buf, val):
    if len(buf.shape) == 1:
        def elt(r, carry):
            buf[pl.ds(r * 16, 16)] = jnp.full((16,), val, _f32)
            return carry
        lax.fori_loop(0, buf.shape[0] // 16, elt, 0)
        return
    rows, cols = buf.shape
    def row(r, carry):
        for k in range(cols // 16):
            buf[r, pl.ds(k * 16, 16)] = jnp.full((16,), val, _f32)
        return carry
    lax.fori_loop(0, rows, row, 0)


def _zero_acc(acc, zbuf, t0):
    nfull, rem = ROWS_PT // C, ROWS_PT % C
    def cp(j, carry):
        pltpu.sync_copy(zbuf, acc.at[pl.ds(t0 + j * C, C)])
        return carry
    lax.fori_loop(0, nfull, cp, 0)
    if rem:
        pltpu.sync_copy(zbuf.at[pl.ds(0, rem)],
                        acc.at[pl.ds(t0 + nfull * C, rem)])


def _unpack_dst(islab, j, db):
    for k in range(C // 16):
        p = islab[j, pl.ds(k * 16, 16)]
        db[pl.ds(k * 16, 16)] = lax.shift_right_logical(p, 16)


def _unpack_src(islab, j, sb):
    for k in range(C // 16):
        p = islab[j, pl.ds(k * 16, 16)]
        sb[pl.ds(k * 16, 16)] = lax.bitwise_and(p, 0xFFFF)


def _deg_body(sd_hbm, out_hbm, acc, buf, islab, dbuf, d0, d1, d2, d3, isem,
              s0, s1, s2, s3):
    c = lax.axis_index("c")
    s = lax.axis_index("s")
    wid = s * NC + c
    t0 = s * ROWS_PT
    sems = [s0, s1, s2, s3]
    didx = [d0, d1, d2, d3]
    idesc = pltpu.async_copy(sd_hbm.at[pl.ds(wid * CPW, CPW)], islab, isem)
    _fill(buf, 0.0)
    _zero_acc(acc, buf, t0)
    _fill(buf, 1.0)
    plsc.subcore_barrier()
    idesc.wait()
    for b in range(4):
        _unpack_dst(islab, b, didx[b])
        pltpu.async_copy(buf, acc.at[didx[b]], sems[b], add=True)

    def outer(i, carry):
        for b in range(4):
            j = i * 4 + b
            pltpu.make_async_copy(buf, acc.at[didx[b]], sems[b]).wait()
            jn = j + 4

            @pl.when(jn < CPW)
            def _next():
                _unpack_dst(islab, jn, didx[b])
                pltpu.async_copy(buf, acc.at[didx[b]], sems[b], add=True)
        return carry
    lax.fori_loop(0, CPW // 4, outer, 0)
    plsc.subcore_barrier()
    @pl.when(s < NP // 1024)
    def _emit():
        for r in range(8):
            pltpu.sync_copy(acc.at[pl.ds(s * 1024 + r * 128, 128)],
                            dbuf.at[r])
        pltpu.sync_copy(dbuf, out_hbm.at[pl.ds(c * (NP // 128) + s * 8, 8)])


_deg_call = pl.kernel(
    _deg_body,
    out_type=jax.ShapeDtypeStruct((NC * NP // 128, 128), _f32),
    mesh=_MESH,
    scratch_types=[
        pltpu.VMEM_SHARED((NP,), _f32),
        pltpu.VMEM((C,), _f32),
        pltpu.VMEM((CPW, C), jnp.int32),
        pltpu.VMEM((8, 128), _f32),
        pltpu.VMEM((C,), jnp.int32),
        pltpu.VMEM((C,), jnp.int32),
        pltpu.VMEM((C,), jnp.int32),
        pltpu.VMEM((C,), jnp.int32),
        pltpu.SemaphoreType.DMA,
        pltpu.SemaphoreType.DMA,
        pltpu.SemaphoreType.DMA,
        pltpu.SemaphoreType.DMA,
        pltpu.SemaphoreType.DMA,
    ],
)


def _agg_body(g_hbm, sd_hbm, out_hbm, acc, r0, r1, islab,
              si0, si1, di0, di1, isem, gs0, gs1, ss0, ss1):
    c = lax.axis_index("c")
    s = lax.axis_index("s")
    wid = s * NC + c
    t0 = s * ROWS_PT
    rowss = [r0, r1]
    sidx = [si0, si1]
    didx = [di0, di1]
    gsems = [gs0, gs1]
    ssems = [ss0, ss1]
    idesc = pltpu.async_copy(sd_hbm.at[pl.ds(wid * CPW, CPW)], islab, isem)
    _fill(r0, 0.0)
    _zero_acc(acc, r0, t0)
    plsc.subcore_barrier()
    idesc.wait()
    for b in range(2):
        _unpack_src(islab, b, sidx[b])
        _unpack_dst(islab, b, didx[b])
        pltpu.async_copy(g_hbm.at[sidx[b]], rowss[b], gsems[b])

    def outer(i, carry):
        for b in range(2):
            j = i * 2 + b
            pltpu.make_async_copy(
                g_hbm.at[sidx[b]], rowss[b], gsems[b]).wait()
            pltpu.async_copy(
                rowss[b], acc.at[didx[b]], ssems[b], add=True)
            jn = j + 2

            @pl.when(jn < CPW)
            def _prefetch():
                pltpu.make_async_copy(
                    rowss[b], acc.at[didx[b]], ssems[b]).wait()
                _unpack_src(islab, jn, sidx[b])
                _unpack_dst(islab, jn, didx[b])
                pltpu.async_copy(g_hbm.at[sidx[b]], rowss[b], gsems[b])
        return carry
    lax.fori_loop(0, CPW // 2, outer, 0)
    pltpu.make_async_copy(rowss[0], acc.at[didx[0]], ssems[0]).wait()
    pltpu.make_async_copy(rowss[1], acc.at[didx[1]], ssems[1]).wait()
    plsc.subcore_barrier()
    pltpu.sync_copy(acc.at[pl.ds(t0, ROWS_PT)],
                    out_hbm.at[c, pl.ds(t0, ROWS_PT)])


_agg_call = pl.kernel(
    _agg_body,
    out_type=jax.ShapeDtypeStruct((NC, NP, H), _f32),
    mesh=_MESH,
    scratch_types=[
        pltpu.VMEM_SHARED((NP, H), _f32),
        pltpu.VMEM((C, H), _f32),
        pltpu.VMEM((C, H), _f32),
        pltpu.VMEM((CPW, C), jnp.int32),
        pltpu.VMEM((C,), jnp.int32),
        pltpu.VMEM((C,), jnp.int32),
        pltpu.VMEM((C,), jnp.int32),
        pltpu.VMEM((C,), jnp.int32),
        pltpu.SemaphoreType.DMA,
        pltpu.SemaphoreType.DMA,
        pltpu.SemaphoreType.DMA,
        pltpu.SemaphoreType.DMA,
        pltpu.SemaphoreType.DMA,
    ],
)


def _k2_body(x_ref, w_ref, dp0_ref, dp1_ref, g_ref, dis_ref):
    v = dp0_ref[...] + dp1_ref[...]
    n_iota = lax.broadcasted_iota(jnp.int32, (BN, 128), 0)
    l_iota = lax.broadcasted_iota(jnp.int32, (BN, 128), 1)
    brd = (l_iota[:, :BN // 128] == n_iota[:, :BN // 128] // 128)
    t = jnp.dot(brd.astype(_f32), v, preferred_element_type=_f32)
    sel = (l_iota == n_iota % 128).astype(_f32)
    deg = jnp.sum(t * sel, axis=1, keepdims=True) + 1.0
    dis = lax.rsqrt(deg)
    m = jnp.dot(x_ref[...], w_ref[...], preferred_element_type=_f32)
    g_ref[...] = dis * m
    dis_ref[...] = jnp.broadcast_to(dis, (BN, 16))


_k2_call = pl.pallas_call(
    _k2_body,
    grid=(NGRID,),
    in_specs=[
        pl.BlockSpec((BN, D), lambda i: (i, 0)),
        pl.BlockSpec((D, H), lambda i: (0, 0)),
        pl.BlockSpec((BN // 128, 128), lambda i: (i, 0)),
        pl.BlockSpec((BN // 128, 128), lambda i: (i + NP // BN, 0)),
    ],
    out_specs=[
        pl.BlockSpec((BN, H), lambda i: (i, 0)),
        pl.BlockSpec((BN, 16), lambda i: (i, 0)),
    ],
    out_shape=[
        jax.ShapeDtypeStruct((NP, H), _f32),
        jax.ShapeDtypeStruct((NP, 16), _f32),
    ],
)


def _k4_body(a_ref, g1_ref, dis_ref, b1_ref, w2_ref, g2_ref):
    dis = dis_ref[:, 0:1]
    t = dis * (a_ref[0] + a_ref[1] + g1_ref[...]) + b1_ref[...]
    h1 = jnp.maximum(t, 0.0)
    g2_ref[...] = dis * jnp.dot(h1, w2_ref[...], preferred_element_type=_f32)


_k4_call = pl.pallas_call(
    _k4_body,
    grid=(NGRID,),
    in_specs=[
        pl.BlockSpec((NC, BN, H), lambda i: (0, i, 0)),
        pl.BlockSpec((BN, H), lambda i: (i, 0)),
        pl.BlockSpec((BN, 16), lambda i: (i, 0)),
        pl.BlockSpec((1, H), lambda i: (0, 0)),
        pl.BlockSpec((H, H), lambda i: (0, 0)),
    ],
    out_specs=pl.BlockSpec((BN, H), lambda i: (i, 0)),
    out_shape=jax.ShapeDtypeStruct((NP, H), _f32),
)


def _k6_body(a_ref, g2_ref, dis_ref, b2_ref, batch_ref, wf1_ref, bf1_ref,
             wf2_ref, bf2_ref, out_ref, sums, cnts):
    i = pl.program_id(0)

    @pl.when(i == 0)
    def _init():
        sums[...] = jnp.zeros((G, H), _f32)
        cnts[...] = jnp.zeros((G, H), _f32)

    dis = dis_ref[:, 0:1]
    t = dis * (a_ref[0] + a_ref[1] + g2_ref[...]) + b2_ref[...]
    h2 = jnp.maximum(t, 0.0)
    gid = lax.broadcasted_iota(jnp.int32, (G, BN), 0)
    mask = (gid == batch_ref[...]).astype(_f32)
    sums[...] += jnp.dot(mask, h2, preferred_element_type=_f32)
    cnts[...] += jnp.broadcast_to(
        jnp.sum(mask, axis=1, keepdims=True), (G, H))

    @pl.when(i == NGRID - 1)
    def _fin():
        pooled = sums[...] / jnp.maximum(cnts[...], 1.0)
        fc1 = jnp.maximum(
            jnp.dot(pooled, wf1_ref[...], preferred_element_type=_f32)
            + bf1_ref[...], 0.0)
        logits = (jnp.dot(fc1, wf2_ref[...], preferred_element_type=_f32)
                  + bf2_ref[...])
        m = jnp.max(logits, axis=1, keepdims=True)
        e = jnp.exp(logits - m)
        out_ref[...] = e / jnp.sum(e, axis=1, keepdims=True)


_k6_call = pl.pallas_call(
    _k6_body,
    grid=(NGRID,),
    in_specs=[
        pl.BlockSpec((NC, BN, H), lambda i: (0, i, 0)),
        pl.BlockSpec((BN, H), lambda i: (i, 0)),
        pl.BlockSpec((BN, 16), lambda i: (i, 0)),
        pl.BlockSpec((1, H), lambda i: (0, 0)),
        pl.BlockSpec((1, BN), lambda i: (0, i)),
        pl.BlockSpec((H, H), lambda i: (0, 0)),
        pl.BlockSpec((1, H), lambda i: (0, 0)),
        pl.BlockSpec((H, O_), lambda i: (0, 0)),
        pl.BlockSpec((1, O_), lambda i: (0, 0)),
    ],
    out_specs=pl.BlockSpec((G, O_), lambda i: (0, 0)),
    out_shape=jax.ShapeDtypeStruct((G, O_), _f32),
    scratch_shapes=[
        pltpu.VMEM((G, H), _f32),
        pltpu.VMEM((G, H), _f32),
    ],
)


def kernel(x, edge_index, batch, W1, b1, W2, b2, Wfc1, bfc1, Wfc2, bfc2):
    src = edge_index[0]
    dst = edge_index[1]
    pad = EP - E0
    pidx = jnp.arange(pad, dtype=jnp.int32)
    srcp = jnp.concatenate([src, (pidx * 97) % N0])
    dstp = jnp.concatenate([dst, N0 + pidx % (NP - N0)])
    sdp = (srcp | (dstp << 16)).reshape(NCH, C)
    xp = jnp.zeros((NP, D), _f32).at[:N0].set(x)
    batchp = jnp.full((1, NP), G, jnp.int32).at[0, :N0].set(batch)

    degp = _deg_call(sdp)
    g1, dis16 = _k2_call(xp, W1, degp, degp)
    agg1 = _agg_call(g1, sdp)
    g2 = _k4_call(agg1, g1, dis16, b1.reshape(1, H), W2)
    agg2 = _agg_call(g2, sdp)
    return _k6_call(agg2, g2, dis16, b2.reshape(1, H), batchp,
                    Wfc1, bfc1.reshape(1, H), Wfc2, bfc2.reshape(1, O_))


# speedup vs baseline: 36.1779x; 1.0219x over previous
import jax
import jax.numpy as jnp
from jax import lax
from jax.experimental import pallas as pl
from jax.experimental.pallas import tpu as pltpu
from jax.experimental.pallas import tpu_sc as plsc

N0 = 10000
E0 = 320000
D = 128
H = 128
O_ = 10
G = 64

NC, NS = 2, 16
NW = NC * NS
BN = 2048
NP = 10240
ROWS_PT = NP // NS
C = 128
CPW = 80
EP = NW * CPW * C
NCH = EP // C
NGRID = NP // BN

_f32 = jnp.float32
_MESH = plsc.VectorSubcoreMesh(core_axis_name="c", subcore_axis_name="s")


def _fill(buf, val):
    if len(buf.shape) == 1:
        def elt(r, carry):
            buf[pl.ds(r * 16, 16)] = jnp.full((16,), val, _f32)
            return carry
        lax.fori_loop(0, buf.shape[0] // 16, elt, 0)
        return
    rows, cols = buf.shape
    def row(r, carry):
        for k in range(cols // 16):
            buf[r, pl.ds(k * 16, 16)] = jnp.full((16,), val, _f32)
        return carry
    lax.fori_loop(0, rows, row, 0)


def _zero_acc(acc, zbuf, t0):
    nfull, rem = ROWS_PT // C, ROWS_PT % C
    def cp(j, carry):
        pltpu.sync_copy(zbuf, acc.at[pl.ds(t0 + j * C, C)])
        return carry
    lax.fori_loop(0, nfull, cp, 0)
    if rem:
        pltpu.sync_copy(zbuf.at[pl.ds(0, rem)],
                        acc.at[pl.ds(t0 + nfull * C, rem)])


def _unpack_dst(islab, j, db):
    for k in range(C // 16):
        p = islab[j, pl.ds(k * 16, 16)]
        db[pl.ds(k * 16, 16)] = lax.shift_right_logical(p, 16)


def _unpack_src(islab, j, sb):
    for k in range(C // 16):
        p = islab[j, pl.ds(k * 16, 16)]
        sb[pl.ds(k * 16, 16)] = lax.bitwise_and(p, 0xFFFF)


def _deg_body(sd_hbm, out_hbm, acc, buf, islab, dbuf, d0, d1, d2, d3, isem,
              s0, s1, s2, s3):
    c = lax.axis_index("c")
    s = lax.axis_index("s")
    wid = s * NC + c
    t0 = s * ROWS_PT
    sems = [s0, s1, s2, s3]
    didx = [d0, d1, d2, d3]
    idesc = pltpu.async_copy(sd_hbm.at[pl.ds(wid * CPW, CPW)], islab, isem)
    _fill(buf, 0.0)
    _zero_acc(acc, buf, t0)
    _fill(buf, 1.0)
    plsc.subcore_barrier()
    idesc.wait()
    for b in range(4):
        _unpack_dst(islab, b, didx[b])
        pltpu.async_copy(buf, acc.at[didx[b]], sems[b], add=True)

    def outer(i, carry):
        for b in range(4):
            j = i * 4 + b
            pltpu.make_async_copy(buf, acc.at[didx[b]], sems[b]).wait()
            jn = j + 4

            @pl.when(jn < CPW)
            def _next():
                _unpack_dst(islab, jn, didx[b])
                pltpu.async_copy(buf, acc.at[didx[b]], sems[b], add=True)
        return carry
    lax.fori_loop(0, CPW // 4, outer, 0)
    plsc.subcore_barrier()
    @pl.when(s < NP // 1024)
    def _emit():
        for r in range(8):
            pltpu.sync_copy(acc.at[pl.ds(s * 1024 + r * 128, 128)],
                            dbuf.at[r])
        pltpu.sync_copy(dbuf, out_hbm.at[pl.ds(c * (NP // 128) + s * 8, 8)])


_deg_call = pl.kernel(
    _deg_body,
    out_type=jax.ShapeDtypeStruct((NC * NP // 128, 128), _f32),
    mesh=_MESH,
    scratch_types=[
        pltpu.VMEM_SHARED((NP,), _f32),
        pltpu.VMEM((C,), _f32),
        pltpu.VMEM((CPW, C), jnp.int32),
        pltpu.VMEM((8, 128), _f32),
        pltpu.VMEM((C,), jnp.int32),
        pltpu.VMEM((C,), jnp.int32),
        pltpu.VMEM((C,), jnp.int32),
        pltpu.VMEM((C,), jnp.int32),
        pltpu.SemaphoreType.DMA,
        pltpu.SemaphoreType.DMA,
        pltpu.SemaphoreType.DMA,
        pltpu.SemaphoreType.DMA,
        pltpu.SemaphoreType.DMA,
    ],
)


def _agg_body(g_hbm, sd_hbm, out_hbm, acc, r0, r1, islab,
              si0, si1, di0, di1, isem, gs0, gs1, ss0, ss1):
    c = lax.axis_index("c")
    s = lax.axis_index("s")
    wid = s * NC + c
    t0 = s * ROWS_PT
    rowss = [r0, r1]
    sidx = [si0, si1]
    didx = [di0, di1]
    gsems = [gs0, gs1]
    ssems = [ss0, ss1]
    idesc = pltpu.async_copy(sd_hbm.at[pl.ds(wid * CPW, CPW)], islab, isem)
    _fill(r0, 0.0)
    _zero_acc(acc, r0, t0)
    plsc.subcore_barrier()
    idesc.wait()
    for b in range(2):
        _unpack_src(islab, b, sidx[b])
        _unpack_dst(islab, b, didx[b])
        pltpu.async_copy(g_hbm.at[sidx[b]], rowss[b], gsems[b])

    def outer(i, carry):
        for b in range(2):
            j = i * 2 + b
            pltpu.make_async_copy(
                g_hbm.at[sidx[b]], rowss[b], gsems[b]).wait()
            pltpu.async_copy(
                rowss[b], acc.at[didx[b]], ssems[b], add=True)
            jn = j + 2

            @pl.when(jn < CPW)
            def _prefetch():
                pltpu.make_async_copy(
                    rowss[b], acc.at[didx[b]], ssems[b]).wait()
                _unpack_src(islab, jn, sidx[b])
                _unpack_dst(islab, jn, didx[b])
                pltpu.async_copy(g_hbm.at[sidx[b]], rowss[b], gsems[b])
        return carry
    lax.fori_loop(0, CPW // 2, outer, 0)
    pltpu.make_async_copy(rowss[0], acc.at[didx[0]], ssems[0]).wait()
    pltpu.make_async_copy(rowss[1], acc.at[didx[1]], ssems[1]).wait()
    plsc.subcore_barrier()
    pltpu.sync_copy(acc.at[pl.ds(t0, ROWS_PT)],
                    out_hbm.at[c, pl.ds(t0, ROWS_PT)])


_agg_call = pl.kernel(
    _agg_body,
    out_type=jax.ShapeDtypeStruct((NC, NP, H), _f32),
    mesh=_MESH,
    scratch_types=[
        pltpu.VMEM_SHARED((NP, H), _f32),
        pltpu.VMEM((C, H), _f32),
        pltpu.VMEM((C, H), _f32),
        pltpu.VMEM((CPW, C), jnp.int32),
        pltpu.VMEM((C,), jnp.int32),
        pltpu.VMEM((C,), jnp.int32),
        pltpu.VMEM((C,), jnp.int32),
        pltpu.VMEM((C,), jnp.int32),
        pltpu.SemaphoreType.DMA,
        pltpu.SemaphoreType.DMA,
        pltpu.SemaphoreType.DMA,
        pltpu.SemaphoreType.DMA,
        pltpu.SemaphoreType.DMA,
    ],
)


def _k2_body(x_ref, w_ref, dp0_ref, dp1_ref, g_ref, dis_ref):
    v = dp0_ref[...] + dp1_ref[...]
    n_iota = lax.broadcasted_iota(jnp.int32, (BN, 128), 0)
    l_iota = lax.broadcasted_iota(jnp.int32, (BN, 128), 1)
    brd = (l_iota[:, :BN // 128] == n_iota[:, :BN // 128] // 128)
    t = jnp.dot(brd.astype(_f32), v, preferred_element_type=_f32)
    sel = (l_iota == n_iota % 128).astype(_f32)
    deg = jnp.sum(t * sel, axis=1, keepdims=True) + 1.0
    dis = lax.rsqrt(deg)
    m = jnp.dot(x_ref[...], w_ref[...], preferred_element_type=_f32)
    g_ref[...] = dis * m
    dis_ref[...] = jnp.broadcast_to(dis, (BN, 16))


_k2_call = pl.pallas_call(
    _k2_body,
    grid=(NGRID,),
    in_specs=[
        pl.BlockSpec((BN, D), lambda i: (i, 0)),
        pl.BlockSpec((D, H), lambda i: (0, 0)),
        pl.BlockSpec((BN // 128, 128), lambda i: (i, 0)),
        pl.BlockSpec((BN // 128, 128), lambda i: (i + NP // BN, 0)),
    ],
    out_specs=[
        pl.BlockSpec((BN, H), lambda i: (i, 0)),
        pl.BlockSpec((BN, 16), lambda i: (i, 0)),
    ],
    out_shape=[
        jax.ShapeDtypeStruct((NP, H), _f32),
        jax.ShapeDtypeStruct((NP, 16), _f32),
    ],
)


def _k4_body(a_ref, g1_ref, dis_ref, b1_ref, w2_ref, g2_ref):
    dis = dis_ref[:, 0:1]
    t = dis * (a_ref[0] + a_ref[1] + g1_ref[...]) + b1_ref[...]
    h1 = jnp.maximum(t, 0.0)
    g2_ref[...] = dis * jnp.dot(h1, w2_ref[...], preferred_element_type=_f32)


_k4_call = pl.pallas_call(
    _k4_body,
    grid=(NGRID,),
    in_specs=[
        pl.BlockSpec((NC, BN, H), lambda i: (0, i, 0)),
        pl.BlockSpec((BN, H), lambda i: (i, 0)),
        pl.BlockSpec((BN, 16), lambda i: (i, 0)),
        pl.BlockSpec((1, H), lambda i: (0, 0)),
        pl.BlockSpec((H, H), lambda i: (0, 0)),
    ],
    out_specs=pl.BlockSpec((BN, H), lambda i: (i, 0)),
    out_shape=jax.ShapeDtypeStruct((NP, H), _f32),
)


def _k6_body(a_ref, g2_ref, dis_ref, b2_ref, batch_ref, wf1_ref, bf1_ref,
             wf2_ref, bf2_ref, out_ref, sums, cnts):
    i = pl.program_id(0)

    @pl.when(i == 0)
    def _init():
        sums[...] = jnp.zeros((G, H), _f32)
        cnts[...] = jnp.zeros((G, H), _f32)

    dis = dis_ref[:, 0:1]
    t = dis * (a_ref[0] + a_ref[1] + g2_ref[...]) + b2_ref[...]
    h2 = jnp.maximum(t, 0.0)
    gid = lax.broadcasted_iota(jnp.int32, (G, BN), 0)
    mask = (gid == batch_ref[...]).astype(_f32)
    sums[...] += jnp.dot(mask, h2, preferred_element_type=_f32)
    cnts[...] += jnp.broadcast_to(
        jnp.sum(mask, axis=1, keepdims=True), (G, H))

    @pl.when(i == NGRID - 1)
    def _fin():
        pooled = sums[...] / jnp.maximum(cnts[...], 1.0)
        fc1 = jnp.maximum(
            jnp.dot(pooled, wf1_ref[...], preferred_element_type=_f32)
            + bf1_ref[...], 0.0)
        logits = (jnp.dot(fc1, wf2_ref[...], preferred_element_type=_f32)
                  + bf2_ref[...])
        m = jnp.max(logits, axis=1, keepdims=True)
        e = jnp.exp(logits - m)
        out_ref[...] = e / jnp.sum(e, axis=1, keepdims=True)


_k6_call = pl.pallas_call(
    _k6_body,
    grid=(NGRID,),
    in_specs=[
        pl.BlockSpec((NC, BN, H), lambda i: (0, i, 0)),
        pl.BlockSpec((BN, H), lambda i: (i, 0)),
        pl.BlockSpec((BN, 16), lambda i: (i, 0)),
        pl.BlockSpec((1, H), lambda i: (0, 0)),
        pl.BlockSpec((1, BN), lambda i: (0, i)),
        pl.BlockSpec((H, H), lambda i: (0, 0)),
        pl.BlockSpec((1, H), lambda i: (0, 0)),
        pl.BlockSpec((H, O_), lambda i: (0, 0)),
        pl.BlockSpec((1, O_), lambda i: (0, 0)),
    ],
    out_specs=pl.BlockSpec((G, O_), lambda i: (0, 0)),
    out_shape=jax.ShapeDtypeStruct((G, O_), _f32),
    scratch_shapes=[
        pltpu.VMEM((G, H), _f32),
        pltpu.VMEM((G, H), _f32),
    ],
)


def kernel(x, edge_index, batch, W1, b1, W2, b2, Wfc1, bfc1, Wfc2, bfc2):
    src = edge_index[0]
    dst = edge_index[1]
    pad = EP - E0
    pidx = jnp.arange(pad, dtype=jnp.int32)
    srcp = jnp.concatenate([src, (pidx * 97) % N0])
    dstp = jnp.concatenate([dst, N0 + pidx % (NP - N0)])
    sdp = (srcp | (dstp << 16)).reshape(NCH, C)
    xp = jnp.zeros((NP, D), _f32).at[:N0].set(x)
    batchp = jnp.full((1, NP), G, jnp.int32).at[0, :N0].set(batch)

    degp = _deg_call(sdp)
    g1, dis16 = _k2_call(xp, W1, degp, degp)
    agg1 = _agg_call(g1, sdp)
    g2 = _k4_call(agg1, g1, dis16, b1.reshape(1, H), W2)
    agg2 = _agg_call(g2, sdp)
    return _k6_call(agg2, g2, dis16, b2.reshape(1, H), batchp,
                    Wfc1, bfc1.reshape(1, H), Wfc2, bfc2.reshape(1, O_))


# speedup vs baseline: 36.3415x; 1.0045x over previous
import jax
import jax.numpy as jnp
from jax import lax
from jax.experimental import pallas as pl
from jax.experimental.pallas import tpu as pltpu
from jax.experimental.pallas import tpu_sc as plsc

N0 = 10000
E0 = 320000
D = 128
H = 128
O_ = 10
G = 64

NC, NS = 2, 16
NW = NC * NS
BN = 2048
NP = 10240
ROWS_PT = NP // NS
C = 128
EPW = E0 // NW
CF = EPW // C
CT = EPW - CF * C
CPW = 80
NCH = NW * CPW
NGRID = NP // BN

_f32 = jnp.float32
_MESH = plsc.VectorSubcoreMesh(core_axis_name="c", subcore_axis_name="s")


def _fill(buf, val):
    if len(buf.shape) == 1:
        def elt(r, carry):
            buf[pl.ds(r * 16, 16)] = jnp.full((16,), val, _f32)
            return carry
        lax.fori_loop(0, buf.shape[0] // 16, elt, 0)
        return
    rows, cols = buf.shape
    def row(r, carry):
        for k in range(cols // 16):
            buf[r, pl.ds(k * 16, 16)] = jnp.full((16,), val, _f32)
        return carry
    lax.fori_loop(0, rows, row, 0)


def _zero_acc(acc, zbuf, t0):
    nfull, rem = ROWS_PT // C, ROWS_PT % C
    def cp(j, carry):
        pltpu.sync_copy(zbuf, acc.at[pl.ds(t0 + j * C, C)])
        return carry
    lax.fori_loop(0, nfull, cp, 0)
    if rem:
        pltpu.sync_copy(zbuf.at[pl.ds(0, rem)],
                        acc.at[pl.ds(t0 + nfull * C, rem)])


def _unpack_dst(islab, j, db, n=C):
    for k in range(n // 16):
        p = islab[j, pl.ds(k * 16, 16)]
        db[pl.ds(k * 16, 16)] = lax.shift_right_logical(p, 16)


def _unpack_src(islab, j, sb, n=C):
    for k in range(n // 16):
        p = islab[j, pl.ds(k * 16, 16)]
        sb[pl.ds(k * 16, 16)] = lax.bitwise_and(p, 0xFFFF)


def _deg_body(sd_hbm, out_hbm, acc, buf, islab, dbuf, d0, d1, d2, d3,
              didx_t, isem, s0, s1, s2, s3):
    c = lax.axis_index("c")
    s = lax.axis_index("s")
    wid = s * NC + c
    t0 = s * ROWS_PT
    sems = [s0, s1, s2, s3]
    didx = [d0, d1, d2, d3]
    idesc = pltpu.async_copy(sd_hbm.at[pl.ds(wid * CPW, CPW)], islab, isem)
    _fill(buf, 0.0)
    _zero_acc(acc, buf, t0)
    _fill(buf, 1.0)
    plsc.subcore_barrier()
    idesc.wait()
    for b in range(4):
        _unpack_dst(islab, b, didx[b])
        pltpu.async_copy(buf, acc.at[didx[b]], sems[b], add=True)

    def outer(i, carry):
        for b in range(4):
            j = i * 4 + b
            pltpu.make_async_copy(buf, acc.at[didx[b]], sems[b]).wait()
            jn = j + 4

            @pl.when(jn < CF)
            def _next():
                _unpack_dst(islab, jn, didx[b])
                pltpu.async_copy(buf, acc.at[didx[b]], sems[b], add=True)
        return carry
    lax.fori_loop(0, CF // 4, outer, 0)
    for b in range(CF % 4):
        pltpu.make_async_copy(buf, acc.at[didx[b]], sems[b]).wait()
    _unpack_dst(islab, CF, didx_t, CT)
    pltpu.sync_copy(buf.at[pl.ds(0, CT)], acc.at[didx_t], add=True)
    plsc.subcore_barrier()
    @pl.when(s < NP // 1024)
    def _emit():
        for r in range(8):
            pltpu.sync_copy(acc.at[pl.ds(s * 1024 + r * 128, 128)],
                            dbuf.at[r])
        pltpu.sync_copy(dbuf, out_hbm.at[pl.ds(c * (NP // 128) + s * 8, 8)])


_deg_call = pl.kernel(
    _deg_body,
    out_type=jax.ShapeDtypeStruct((NC * NP // 128, 128), _f32),
    mesh=_MESH,
    scratch_types=[
        pltpu.VMEM_SHARED((NP,), _f32),
        pltpu.VMEM((C,), _f32),
        pltpu.VMEM((CPW, C), jnp.int32),
        pltpu.VMEM((8, 128), _f32),
        pltpu.VMEM((C,), jnp.int32),
        pltpu.VMEM((C,), jnp.int32),
        pltpu.VMEM((C,), jnp.int32),
        pltpu.VMEM((C,), jnp.int32),
        pltpu.VMEM((CT,), jnp.int32),
        pltpu.SemaphoreType.DMA,
        pltpu.SemaphoreType.DMA,
        pltpu.SemaphoreType.DMA,
        pltpu.SemaphoreType.DMA,
        pltpu.SemaphoreType.DMA,
    ],
)


def _agg_body(g_hbm, sd_hbm, out_hbm, acc, r0, r1, islab,
              si0, si1, di0, di1, sidx_t, didx_t, rows_t,
              isem, gs0, gs1, ss0, ss1):
    c = lax.axis_index("c")
    s = lax.axis_index("s")
    wid = s * NC + c
    t0 = s * ROWS_PT
    rowss = [r0, r1]
    sidx = [si0, si1]
    didx = [di0, di1]
    gsems = [gs0, gs1]
    ssems = [ss0, ss1]
    idesc = pltpu.async_copy(sd_hbm.at[pl.ds(wid * CPW, CPW)], islab, isem)
    _fill(r0, 0.0)
    _zero_acc(acc, r0, t0)
    plsc.subcore_barrier()
    idesc.wait()
    for b in range(2):
        _unpack_src(islab, b, sidx[b])
        _unpack_dst(islab, b, didx[b])
        pltpu.async_copy(g_hbm.at[sidx[b]], rowss[b], gsems[b])

    def outer(i, carry):
        for b in range(2):
            j = i * 2 + b
            pltpu.make_async_copy(
                g_hbm.at[sidx[b]], rowss[b], gsems[b]).wait()
            pltpu.async_copy(
                rowss[b], acc.at[didx[b]], ssems[b], add=True)
            jn = j + 2

            @pl.when(jn < CF)
            def _prefetch():
                pltpu.make_async_copy(
                    rowss[b], acc.at[didx[b]], ssems[b]).wait()
                _unpack_src(islab, jn, sidx[b])
                _unpack_dst(islab, jn, didx[b])
                pltpu.async_copy(g_hbm.at[sidx[b]], rowss[b], gsems[b])
        return carry
    lax.fori_loop(0, CF // 2, outer, 0)
    pltpu.make_async_copy(rowss[0], acc.at[didx[0]], ssems[0]).wait()
    pltpu.make_async_copy(rowss[1], acc.at[didx[1]], ssems[1]).wait()
    _unpack_src(islab, CF, sidx_t, CT)
    _unpack_dst(islab, CF, didx_t, CT)
    pltpu.async_copy(g_hbm.at[sidx_t], rows_t, gs0).wait()
    pltpu.sync_copy(rows_t, acc.at[didx_t], add=True)
    plsc.subcore_barrier()
    pltpu.sync_copy(acc.at[pl.ds(t0, ROWS_PT)],
                    out_hbm.at[c, pl.ds(t0, ROWS_PT)])


_agg_call = pl.kernel(
    _agg_body,
    out_type=jax.ShapeDtypeStruct((NC, NP, H), _f32),
    mesh=_MESH,
    scratch_types=[
        pltpu.VMEM_SHARED((NP, H), _f32),
        pltpu.VMEM((C, H), _f32),
        pltpu.VMEM((C, H), _f32),
        pltpu.VMEM((CPW, C), jnp.int32),
        pltpu.VMEM((C,), jnp.int32),
        pltpu.VMEM((C,), jnp.int32),
        pltpu.VMEM((C,), jnp.int32),
        pltpu.VMEM((C,), jnp.int32),
        pltpu.VMEM((CT,), jnp.int32),
        pltpu.VMEM((CT,), jnp.int32),
        pltpu.VMEM((CT, H), _f32),
        pltpu.SemaphoreType.DMA,
        pltpu.SemaphoreType.DMA,
        pltpu.SemaphoreType.DMA,
        pltpu.SemaphoreType.DMA,
        pltpu.SemaphoreType.DMA,
    ],
)


def _k2_body(x_ref, w_ref, dp0_ref, dp1_ref, g_ref, dis_ref):
    v = dp0_ref[...] + dp1_ref[...]
    n_iota = lax.broadcasted_iota(jnp.int32, (BN, 128), 0)
    l_iota = lax.broadcasted_iota(jnp.int32, (BN, 128), 1)
    brd = (l_iota[:, :BN // 128] == n_iota[:, :BN // 128] // 128)
    t = jnp.dot(brd.astype(_f32), v, preferred_element_type=_f32)
    sel = (l_iota == n_iota % 128).astype(_f32)
    deg = jnp.sum(t * sel, axis=1, keepdims=True) + 1.0
    dis = lax.rsqrt(deg)
    m = jnp.dot(x_ref[...], w_ref[...], preferred_element_type=_f32)
    g_ref[...] = dis * m
    dis_ref[...] = jnp.broadcast_to(dis, (BN, 16))


_k2_call = pl.pallas_call(
    _k2_body,
    grid=(NGRID,),
    in_specs=[
        pl.BlockSpec((BN, D), lambda i: (i, 0)),
        pl.BlockSpec((D, H), lambda i: (0, 0)),
        pl.BlockSpec((BN // 128, 128), lambda i: (i, 0)),
        pl.BlockSpec((BN // 128, 128), lambda i: (i + NP // BN, 0)),
    ],
    out_specs=[
        pl.BlockSpec((BN, H), lambda i: (i, 0)),
        pl.BlockSpec((BN, 16), lambda i: (i, 0)),
    ],
    out_shape=[
        jax.ShapeDtypeStruct((NP, H), _f32),
        jax.ShapeDtypeStruct((NP, 16), _f32),
    ],
)


def _k4_body(a_ref, g1_ref, dis_ref, b1_ref, w2_ref, g2_ref):
    dis = dis_ref[:, 0:1]
    t = dis * (a_ref[0] + a_ref[1] + g1_ref[...]) + b1_ref[...]
    h1 = jnp.maximum(t, 0.0)
    g2_ref[...] = dis * jnp.dot(h1, w2_ref[...], preferred_element_type=_f32)


_k4_call = pl.pallas_call(
    _k4_body,
    grid=(NGRID,),
    in_specs=[
        pl.BlockSpec((NC, BN, H), lambda i: (0, i, 0)),
        pl.BlockSpec((BN, H), lambda i: (i, 0)),
        pl.BlockSpec((BN, 16), lambda i: (i, 0)),
        pl.BlockSpec((1, H), lambda i: (0, 0)),
        pl.BlockSpec((H, H), lambda i: (0, 0)),
    ],
    out_specs=pl.BlockSpec((BN, H), lambda i: (i, 0)),
    out_shape=jax.ShapeDtypeStruct((NP, H), _f32),
)


def _k6_body(a_ref, g2_ref, dis_ref, b2_ref, batch_ref, wf1_ref, bf1_ref,
             wf2_ref, bf2_ref, out_ref, sums, cnts):
    i = pl.program_id(0)

    @pl.when(i == 0)
    def _init():
        sums[...] = jnp.zeros((G, H), _f32)
        cnts[...] = jnp.zeros((G, H), _f32)

    dis = dis_ref[:, 0:1]
    t = dis * (a_ref[0] + a_ref[1] + g2_ref[...]) + b2_ref[...]
    h2 = jnp.maximum(t, 0.0)
    gid = lax.broadcasted_iota(jnp.int32, (G, BN), 0)
    mask = (gid == batch_ref[...]).astype(_f32)
    sums[...] += jnp.dot(mask, h2, preferred_element_type=_f32)
    cnts[...] += jnp.broadcast_to(
        jnp.sum(mask, axis=1, keepdims=True), (G, H))

    @pl.when(i == NGRID - 1)
    def _fin():
        pooled = sums[...] / jnp.maximum(cnts[...], 1.0)
        fc1 = jnp.maximum(
            jnp.dot(pooled, wf1_ref[...], preferred_element_type=_f32)
            + bf1_ref[...], 0.0)
        logits = (jnp.dot(fc1, wf2_ref[...], preferred_element_type=_f32)
                  + bf2_ref[...])
        m = jnp.max(logits, axis=1, keepdims=True)
        e = jnp.exp(logits - m)
        out_ref[...] = e / jnp.sum(e, axis=1, keepdims=True)


_k6_call = pl.pallas_call(
    _k6_body,
    grid=(NGRID,),
    in_specs=[
        pl.BlockSpec((NC, BN, H), lambda i: (0, i, 0)),
        pl.BlockSpec((BN, H), lambda i: (i, 0)),
        pl.BlockSpec((BN, 16), lambda i: (i, 0)),
        pl.BlockSpec((1, H), lambda i: (0, 0)),
        pl.BlockSpec((1, BN), lambda i: (0, i)),
        pl.BlockSpec((H, H), lambda i: (0, 0)),
        pl.BlockSpec((1, H), lambda i: (0, 0)),
        pl.BlockSpec((H, O_), lambda i: (0, 0)),
        pl.BlockSpec((1, O_), lambda i: (0, 0)),
    ],
    out_specs=pl.BlockSpec((G, O_), lambda i: (0, 0)),
    out_shape=jax.ShapeDtypeStruct((G, O_), _f32),
    scratch_shapes=[
        pltpu.VMEM((G, H), _f32),
        pltpu.VMEM((G, H), _f32),
    ],
)


def kernel(x, edge_index, batch, W1, b1, W2, b2, Wfc1, bfc1, Wfc2, bfc2):
    src = edge_index[0]
    dst = edge_index[1]
    packed = (src | (dst << 16)).reshape(NW, EPW)
    packed = jnp.pad(packed, ((0, 0), (0, CPW * C - EPW)))
    sdp = packed.reshape(NCH, C)
    xp = jnp.zeros((NP, D), _f32).at[:N0].set(x)
    batchp = jnp.full((1, NP), G, jnp.int32).at[0, :N0].set(batch)

    degp = _deg_call(sdp)
    g1, dis16 = _k2_call(xp, W1, degp, degp)
    agg1 = _agg_call(g1, sdp)
    g2 = _k4_call(agg1, g1, dis16, b1.reshape(1, H), W2)
    agg2 = _agg_call(g2, sdp)
    return _k6_call(agg2, g2, dis16, b2.reshape(1, H), batchp,
                    Wfc1, bfc1.reshape(1, H), Wfc2, bfc2.reshape(1, O_))
